# Initial kernel scaffold; baseline (speedup 1.0000x reference)
import jax
import jax.numpy as jnp
from jax.experimental import pallas as pl


def kernel(x, edge_index, node_emb, edge_emb, sage0_Wl, sage0_bl, sage0_Wr, sage1_Wl, sage1_bl, sage1_Wr, tree_Wl, tree_bl, tree_Wr, gate_W, gate_b, featmask_W, featmask_b, mlp_W1, mlp_b1, mlp_W2, mlp_b2, gcn0_W, gcn0_b, gcn1_W, gcn1_b, lin_W, lin_b):
    raise NotImplementedError("write your pallas kernel here")


# speedup vs baseline: 11.2462x; 11.2462x over previous
import functools
import jax
import jax.numpy as jnp
from jax import lax
from jax.experimental import pallas as pl
from jax.experimental.pallas import tpu as pltpu, tpu_sc as plsc

N = 10000
E = 320000
R = 3
LT = 2
D = 128
H = 32
OUT = 2

NC, NS = 2, 16
NW = NC * NS
EW = E // NW
K = 400
NCH = EW // K
RPT = 624
TOFF = NS * RPT
TAIL = N - TOFF

BN = 400
GRID = N // BN

_MESH = plsc.VectorSubcoreMesh(core_axis_name="c", subcore_axis_name="s")
_CP_SC = pltpu.CompilerParams(use_tc_tiling_on_sc=False)


def _wid():
    return lax.axis_index("s") * NC + lax.axis_index("c")


def _make_seg3(S):
    @functools.partial(
        pl.kernel,
        out_type=(
            jax.ShapeDtypeStruct((NC, S, N, H), jnp.float32),
            jax.ShapeDtypeStruct((NC, S, N), jnp.float32),
        ),
        mesh=_MESH,
        compiler_params=_CP_SC,
        scratch_types=[
            pltpu.VMEM((K,), jnp.int32),
            pltpu.VMEM((K,), jnp.int32),
            pltpu.VMEM((K, H), jnp.float32),
            pltpu.VMEM((K,), jnp.float32),
            pltpu.VMEM_SHARED((S, N, H), jnp.float32),
            pltpu.VMEM_SHARED((S, N), jnp.float32),
            pltpu.SemaphoreType.DMA,
        ],
    )
    def seg3(tables, src, dst, znh, zn, out_vals, out_cnt, src_v, dst_v,
             rows_v, ones_v, acc_sh, cnt_sh, sem):
        sid = lax.axis_index("s")
        wid = _wid()
        one16 = jnp.ones((16,), jnp.float32)
        for i in range(K // 16):
            ones_v[pl.ds(16 * i, 16)] = one16
        for s in range(S):
            pltpu.sync_copy(znh.at[pl.ds(sid * RPT, RPT)],
                            acc_sh.at[s, pl.ds(sid * RPT, RPT)])
            pltpu.sync_copy(zn.at[pl.ds(sid * RPT, RPT)],
                            cnt_sh.at[s, pl.ds(sid * RPT, RPT)])

        @pl.when(sid == 0)
        def _():
            for s in range(S):
                pltpu.sync_copy(znh.at[pl.ds(TOFF, TAIL)],
                                acc_sh.at[s, pl.ds(TOFF, TAIL)])
                pltpu.sync_copy(zn.at[pl.ds(TOFF, TAIL)],
                                cnt_sh.at[s, pl.ds(TOFF, TAIL)])

        plsc.subcore_barrier()
        for s in range(S):
            def chunk(j, _):
                base = wid * EW + j * K
                pltpu.sync_copy(src.at[s, pl.ds(base, K)], src_v)
                pltpu.sync_copy(dst.at[s, pl.ds(base, K)], dst_v)
                pltpu.async_copy(tables.at[s].at[src_v], rows_v, sem).wait()
                pltpu.sync_copy(rows_v, acc_sh.at[s].at[dst_v], add=True)
                pltpu.sync_copy(ones_v, cnt_sh.at[s].at[dst_v], add=True)
                return 0
            lax.fori_loop(0, NCH, chunk, 0)
        plsc.subcore_barrier()
        cid = lax.axis_index("c")
        for s in range(S):
            pltpu.sync_copy(acc_sh.at[s, pl.ds(sid * RPT, RPT)],
                            out_vals.at[cid, s, pl.ds(sid * RPT, RPT)])
            pltpu.sync_copy(cnt_sh.at[s, pl.ds(sid * RPT, RPT)],
                            out_cnt.at[cid, s, pl.ds(sid * RPT, RPT)])

        @pl.when(sid == 0)
        def _():
            for s in range(S):
                pltpu.sync_copy(acc_sh.at[s, pl.ds(TOFF, TAIL)],
                                out_vals.at[cid, s, pl.ds(TOFF, TAIL)])
                pltpu.sync_copy(cnt_sh.at[s, pl.ds(TOFF, TAIL)],
                                out_cnt.at[cid, s, pl.ds(TOFF, TAIL)])

    return seg3


_seg3 = _make_seg3(R)


@functools.partial(
    pl.kernel,
    out_type=jax.ShapeDtypeStruct((R, E), jnp.float32),
    mesh=_MESH,
    compiler_params=_CP_SC,
    scratch_types=[
        pltpu.VMEM((K,), jnp.int32),
        pltpu.VMEM((K,), jnp.int32),
        pltpu.VMEM((K, D), jnp.float32),
        pltpu.VMEM((K, D), jnp.float32),
        pltpu.VMEM((R, D), jnp.float32),
        pltpu.VMEM((D,), jnp.float32),
        pltpu.VMEM((K,), jnp.float32),
        pltpu.SemaphoreType.DMA,
    ],
)
def _emk(A, B, src, dst, cvec, w2, out, src_v, dst_v, ra_v, rb_v, c_v, w_v,
         em_v, sem):
    wid = _wid()
    pltpu.sync_copy(cvec, c_v)
    pltpu.sync_copy(w2, w_v)
    lane = lax.iota(jnp.int32, 16)
    ws = [w_v[pl.ds(16 * t, 16)] for t in range(D // 16)]
    for r in range(R):
        cs = [c_v[r, pl.ds(16 * t, 16)] for t in range(D // 16)]

        def chunk(j, _):
            base = wid * EW + j * K
            pltpu.sync_copy(src.at[r, pl.ds(base, K)], src_v)
            pltpu.sync_copy(dst.at[r, pl.ds(base, K)], dst_v)
            pltpu.async_copy(A.at[r].at[src_v], ra_v, sem).wait()
            pltpu.async_copy(B.at[r].at[dst_v], rb_v, sem).wait()

            def group(g, _):
                vec = jnp.zeros((16,), jnp.float32)
                for i in range(16):
                    e = 16 * g + i
                    acc = jnp.zeros((16,), jnp.float32)
                    for t in range(D // 16):
                        a = ra_v[e, pl.ds(16 * t, 16)]
                        b = rb_v[e, pl.ds(16 * t, 16)]
                        acc = acc + jnp.maximum(a + b + cs[t], 0.0) * ws[t]
                    s01 = acc[0] + acc[1]
                    s23 = acc[2] + acc[3]
                    s45 = acc[4] + acc[5]
                    s67 = acc[6] + acc[7]
                    s89 = acc[8] + acc[9]
                    sab = acc[10] + acc[11]
                    scd = acc[12] + acc[13]
                    sef = acc[14] + acc[15]
                    tot = ((s01 + s23) + (s45 + s67)) + ((s89 + sab) + (scd + sef))
                    vec = jnp.where(lane == i, tot, vec)
                em_v[pl.ds(16 * g, 16)] = vec
                return 0

            lax.fori_loop(0, K // 16, group, 0)
            pltpu.sync_copy(em_v, out.at[r, pl.ds(base, K)])
            return 0

        lax.fori_loop(0, NCH, chunk, 0)


@functools.partial(
    pl.kernel,
    out_type=jax.ShapeDtypeStruct((NC, R, N), jnp.float32),
    mesh=_MESH,
    compiler_params=_CP_SC,
    scratch_types=[
        pltpu.VMEM((K,), jnp.int32),
        pltpu.VMEM((K,), jnp.float32),
        pltpu.VMEM((K,), jnp.float32),
        pltpu.VMEM((16,), jnp.float32),
        pltpu.VMEM_SHARED((R, N), jnp.float32),
        pltpu.SemaphoreType.DMA,
    ],
)
def _degk(em, dst, mnmx, zn, out_deg, dst_v, em_v, ew_v, mm_v, cnt_sh, sem):
    sid = lax.axis_index("s")
    wid = _wid()
    pltpu.sync_copy(mnmx, mm_v)
    mm = mm_v[...]
    for r in range(R):
        pltpu.sync_copy(zn.at[pl.ds(sid * RPT, RPT)],
                        cnt_sh.at[r, pl.ds(sid * RPT, RPT)])

    @pl.when(sid == 0)
    def _():
        for r in range(R):
            pltpu.sync_copy(zn.at[pl.ds(TOFF, TAIL)],
                            cnt_sh.at[r, pl.ds(TOFF, TAIL)])

    plsc.subcore_barrier()
    for r in range(R):
        mn = mm[r]
        inv = mm[R + r]

        def chunk(j, _):
            base = wid * EW + j * K
            pltpu.sync_copy(dst.at[r, pl.ds(base, K)], dst_v)
            pltpu.sync_copy(em.at[r, pl.ds(base, K)], em_v)
            for g in range(K // 16):
                ew_v[pl.ds(16 * g, 16)] = (em_v[pl.ds(16 * g, 16)] - mn) * inv
            pltpu.sync_copy(ew_v, cnt_sh.at[r].at[dst_v], add=True)
            return 0

        lax.fori_loop(0, NCH, chunk, 0)
    plsc.subcore_barrier()
    cid = lax.axis_index("c")
    for r in range(R):
        pltpu.sync_copy(cnt_sh.at[r, pl.ds(sid * RPT, RPT)],
                        out_deg.at[cid, r, pl.ds(sid * RPT, RPT)])

    @pl.when(sid == 0)
    def _():
        for r in range(R):
            pltpu.sync_copy(cnt_sh.at[r, pl.ds(TOFF, TAIL)],
                            out_deg.at[cid, r, pl.ds(TOFF, TAIL)])


@functools.partial(
    pl.kernel,
    out_type=jax.ShapeDtypeStruct((NC, R, N, H), jnp.float32),
    mesh=_MESH,
    compiler_params=_CP_SC,
    scratch_types=[
        pltpu.VMEM((K,), jnp.int32),
        pltpu.VMEM((K,), jnp.int32),
        pltpu.VMEM((K,), jnp.float32),
        pltpu.VMEM((K, H), jnp.float32),
        pltpu.VMEM((16,), jnp.float32),
        pltpu.VMEM_SHARED((R, N, H), jnp.float32),
        pltpu.SemaphoreType.DMA,
    ],
)
def _gcnk(tables, src, dst, em, mnmx, znh, out_vals, src_v, dst_v, em_v,
          rows_v, mm_v, acc_sh, sem):
    sid = lax.axis_index("s")
    wid = _wid()
    pltpu.sync_copy(mnmx, mm_v)
    mm = mm_v[...]
    for r in range(R):
        pltpu.sync_copy(znh.at[pl.ds(sid * RPT, RPT)],
                        acc_sh.at[r, pl.ds(sid * RPT, RPT)])

    @pl.when(sid == 0)
    def _():
        for r in range(R):
            pltpu.sync_copy(znh.at[pl.ds(TOFF, TAIL)],
                            acc_sh.at[r, pl.ds(TOFF, TAIL)])

    plsc.subcore_barrier()
    for r in range(R):
        mn = mm[r]
        inv = mm[R + r]

        def chunk(j, _):
            base = wid * EW + j * K
            pltpu.sync_copy(src.at[r, pl.ds(base, K)], src_v)
            pltpu.sync_copy(dst.at[r, pl.ds(base, K)], dst_v)
            pltpu.sync_copy(em.at[r, pl.ds(base, K)], em_v)
            pltpu.async_copy(tables.at[r].at[src_v], rows_v, sem).wait()

            def group(g, _):
                ew16 = (em_v[pl.ds(16 * g, 16)] - mn) * inv
                for i in range(16):
                    e = 16 * g + i
                    w = ew16[i]
                    rows_v[e, pl.ds(0, 16)] = rows_v[e, pl.ds(0, 16)] * w
                    rows_v[e, pl.ds(16, 16)] = rows_v[e, pl.ds(16, 16)] * w
                return 0

            lax.fori_loop(0, K // 16, group, 0)
            pltpu.sync_copy(rows_v, acc_sh.at[r].at[dst_v], add=True)
            return 0

        lax.fori_loop(0, NCH, chunk, 0)
    plsc.subcore_barrier()
    cid = lax.axis_index("c")
    for r in range(R):
        pltpu.sync_copy(acc_sh.at[r, pl.ds(sid * RPT, RPT)],
                        out_vals.at[cid, r, pl.ds(sid * RPT, RPT)])

    @pl.when(sid == 0)
    def _():
        for r in range(R):
            pltpu.sync_copy(acc_sh.at[r, pl.ds(TOFF, TAIL)],
                            out_vals.at[cid, r, pl.ds(TOFF, TAIL)])


def _full(spec_shape):
    nd = len(spec_shape)
    return pl.BlockSpec(spec_shape, lambda i, _n=nd: (0,) * _n)


def _rows(shape_prefix, suffix):
    blk = tuple(shape_prefix) + (BN,) + tuple(suffix)
    npre = len(shape_prefix)
    nsuf = len(suffix)

    def imap(i, _p=npre, _s=nsuf):
        return (0,) * _p + (i,) + (0,) * _s

    return pl.BlockSpec(blk, imap)


def _t0_body(x_ref, w_ref, o_ref):
    o_ref[...] = jnp.dot(x_ref[...], w_ref[...],
                         preferred_element_type=jnp.float32)


def _t1_body(s0p, c0p, st0p, ct0p, st1p, ct1p, xw0r, xwtr0, xwtr1, b0l, tbl,
             gw, gb, w1l, w1r, b1l, l1l_o, lo2b_o):
    for r in range(R):
        cm = jnp.maximum(c0p[0, r] + c0p[1, r], 1.0)
        lo1 = jnp.maximum((s0p[0, r] + s0p[1, r]) / cm + b0l[r][None, :]
                          + xw0r[r], 0.0)
        ct0 = jnp.maximum(ct0p[0, r] + ct0p[1, r], 1.0)
        t0 = jnp.maximum((st0p[0, r] + st0p[1, r]) / ct0
                         + tbl[r, 0][None, :] + xwtr0[r], 0.0)
        ct1 = jnp.maximum(ct1p[0, r] + ct1p[1, r], 1.0)
        t1 = jnp.maximum((st1p[0, r] + st1p[1, r]) / ct1
                         + tbl[r, 1][None, :] + xwtr1[r], 0.0)
        g0 = jnp.dot(t0, gw[r, 0], preferred_element_type=jnp.float32) + gb[r, 0][None, :]
        g1 = jnp.dot(t1, gw[r, 1], preferred_element_type=jnp.float32) + gb[r, 1][None, :]
        m = jnp.maximum(g0, g1)
        e0 = jnp.exp(g0 - m)
        e1 = jnp.exp(g1 - m)
        tot = e0 + e1
        x_tree = t0 * (e0 / tot) + t1 * (e1 / tot)
        l1l_o[r] = jnp.dot(lo1, w1l[r], preferred_element_type=jnp.float32)
        lo2b_o[r] = (jnp.dot(lo1, w1r[r], preferred_element_type=jnp.float32)
                     + b1l[r][None, :] + x_tree)


def _t2_body(s1p, c0p, lo2b, x_ref, xf, w1a, w1b, w1c, b1, flo, fnt, fb, nemb,
             eemb, g0w, a_o, b_o, xw0_o, cvec_o):
    xblk = x_ref[...]
    for r in range(R):
        cm = jnp.maximum(c0p[0, r] + c0p[1, r], 1.0)
        lo = (s1p[0, r] + s1p[1, r]) / cm + lo2b[r]
        a_o[r] = jnp.dot(lo, w1a[...], preferred_element_type=jnp.float32)
        b_o[r] = jnp.dot(lo, w1b[...], preferred_element_type=jnp.float32)
        ntv = (jnp.dot(nemb[r][None, :], fnt[r],
                       preferred_element_type=jnp.float32) + fb[r][None, :])
        fm = (jnp.dot(lo, flo[r], preferred_element_type=jnp.float32)
              + xf[r] + ntv)
        ly = xblk * fm
        xw0_o[r] = jnp.dot(ly, g0w[r], preferred_element_type=jnp.float32)
        cvec_o[r] = (jnp.dot(eemb[r][None, :], w1c[...],
                             preferred_element_type=jnp.float32)[0]
                     + b1[...])


def _t3_body(em_ref, o_ref):
    i = pl.program_id(0)

    @pl.when(i == 0)
    def _():
        o_ref[...] = jnp.full((8, 128), jnp.inf, jnp.float32)

    lanes = lax.broadcasted_iota(jnp.int32, (1, 128), 1)
    for r in range(R):
        mnr = jnp.min(em_ref[r])
        mxn = jnp.min(-em_ref[r])
        o_ref[0:1, :] = jnp.minimum(o_ref[0:1, :],
                                    jnp.where(lanes == r, mnr, jnp.inf))
        o_ref[1:2, :] = jnp.minimum(o_ref[1:2, :],
                                    jnp.where(lanes == r, mxn, jnp.inf))


def _t4_body(wdegp, xw0, dinv_o, xw0p_o):
    deg = wdegp[0] + wdegp[1] + 1.0
    dinv = lax.rsqrt(deg)
    dinv_o[...] = dinv
    xw0p_o[...] = xw0[...] * dinv


def _t5_body(g0p, xw0, dinv, g0b, g1w, xw1_o, xw1p_o):
    for r in range(R):
        dv = dinv[r]
        h0 = jnp.maximum(dv * (g0p[0, r] + g0p[1, r])
                         + xw0[r] * (dv * dv) + g0b[r][None, :], 0.0)
        xw1 = jnp.dot(h0, g1w[r], preferred_element_type=jnp.float32)
        xw1_o[r] = xw1
        xw1p_o[r] = xw1 * dv


def _t6_body(g1p, xw1, dinv, g1b, lw, lb, out_o, xt_o):
    hs = []
    for r in range(R):
        dv = dinv[r]
        hs.append(dv * (g1p[0, r] + g1p[1, r]) + xw1[r] * (dv * dv)
                  + g1b[r][None, :])
    xt = jnp.concatenate(hs, axis=1)
    xt_o[...] = xt
    logits = jnp.dot(xt, lw[...], preferred_element_type=jnp.float32) + lb[...][None, :]
    logits = jnp.clip(logits, -1e10, 1e10)
    m = jnp.max(logits, axis=1, keepdims=True)
    s = logits - m
    lse = jnp.log(jnp.sum(jnp.exp(s), axis=1, keepdims=True))
    out_o[...] = s - lse


def kernel(x, edge_index, node_emb, edge_emb, sage0_Wl, sage0_bl, sage0_Wr,
           sage1_Wl, sage1_bl, sage1_Wr, tree_Wl, tree_bl, tree_Wr, gate_W,
           gate_b, featmask_W, featmask_b, mlp_W1, mlp_b1, mlp_W2, mlp_b2,
           gcn0_W, gcn0_b, gcn1_W, gcn1_b, lin_W, lin_b):
    f32 = jnp.float32
    x = x.astype(f32)
    ei = edge_index.astype(jnp.int32)
    src_main = ei[:, 0, 0]
    dst_main = ei[:, 0, 1]
    src_t0 = ei[:, 1, 0]
    dst_t0 = ei[:, 1, 1]
    src_t1 = ei[:, 2, 0]
    dst_t1 = ei[:, 2, 1]

    znh = jnp.zeros((N, H), f32)
    zn = jnp.zeros((N,), f32)

    w_s0l = sage0_Wl.transpose(1, 0, 2).reshape(D, R * H)
    w_trl = tree_Wl.transpose(2, 0, 1, 3).reshape(D, R * LT * H)
    w_s0r = sage0_Wr.transpose(1, 0, 2).reshape(D, R * H)
    w_trr = tree_Wr.transpose(2, 0, 1, 3).reshape(D, R * LT * H)
    w_fx = featmask_W[:, H:H + D, :].transpose(1, 0, 2).reshape(D, R * D)
    w_all = jnp.concatenate([w_s0l, w_trl, w_s0r, w_trr, w_fx], axis=1)
    CW = w_all.shape[1]

    t0 = pl.pallas_call(
        _t0_body,
        grid=(GRID,),
        in_specs=[_rows((), (D,)), _full((D, CW))],
        out_specs=_rows((), (CW,)),
        out_shape=jax.ShapeDtypeStruct((N, CW), f32),
    )(x, w_all)

    c = 0
    sage0_tab = t0[:, c:c + 96].reshape(N, R, H).transpose(1, 0, 2); c += 96
    tree_tab = t0[:, c:c + 192].reshape(N, R * LT, H).transpose(1, 0, 2); c += 192
    xW0r = t0[:, c:c + 96].reshape(N, R, H).transpose(1, 0, 2); c += 96
    xWtr = t0[:, c:c + 192].reshape(N, R * LT, H).transpose(1, 0, 2); c += 192
    xF = t0[:, c:c + 384].reshape(N, R, D).transpose(1, 0, 2)

    s0p, c0p = _seg3(sage0_tab, src_main, dst_main, znh, zn)
    st0p, ct0p = _seg3(tree_tab[0::2], src_t0, dst_t0, znh, zn)
    st1p, ct1p = _seg3(tree_tab[1::2], src_t1, dst_t1, znh, zn)
    c0p = c0p.reshape(NC, R, N, 1)
    ct0p = ct0p.reshape(NC, R, N, 1)
    ct1p = ct1p.reshape(NC, R, N, 1)

    l1l, lo2b = pl.pallas_call(
        _t1_body,
        grid=(GRID,),
        in_specs=[
            _rows((2, R), (H,)), _rows((2, R), (1,)),
            _rows((2, R), (H,)), _rows((2, R), (1,)),
            _rows((2, R), (H,)), _rows((2, R), (1,)),
            _rows((R,), (H,)), _rows((R,), (H,)), _rows((R,), (H,)),
            _full((R, H)), _full((R, LT, H)),
            _full((R, LT, H, 1)), _full((R, LT, 1)),
            _full((R, H, H)), _full((R, H, H)), _full((R, H)),
        ],
        out_specs=[_rows((R,), (H,)), _rows((R,), (H,))],
        out_shape=[jax.ShapeDtypeStruct((R, N, H), f32),
                   jax.ShapeDtypeStruct((R, N, H), f32)],
    )(s0p, c0p, st0p, ct0p, st1p, ct1p, xW0r, xWtr[0::2], xWtr[1::2],
      sage0_bl, tree_bl, gate_W, gate_b, sage1_Wl, sage1_Wr, sage1_bl)

    s1p, _unused = _seg3(l1l, src_main, dst_main, znh, zn)

    w1a = mlp_W1[0:H]
    w1b = mlp_W1[H:2 * H]
    w1c = mlp_W1[2 * H:3 * H]
    flo = featmask_W[:, 0:H, :]
    fnt = featmask_W[:, H + D:, :]

    A, B, xw0, cvec = pl.pallas_call(
        _t2_body,
        grid=(GRID,),
        in_specs=[
            _rows((2, R), (H,)), _rows((2, R), (1,)),
            _rows((R,), (H,)), _rows((), (D,)), _rows((R,), (D,)),
            _full((H, D)), _full((H, D)), _full((H, D)), _full((D,)),
            _full((R, H, D)), _full((R, H, D)), _full((R, D)),
            _full((R, H)), _full((R, H)), _full((R, D, H)),
        ],
        out_specs=[_rows((R,), (D,)), _rows((R,), (D,)), _rows((R,), (H,)),
                   _full((R, D))],
        out_shape=[jax.ShapeDtypeStruct((R, N, D), f32),
                   jax.ShapeDtypeStruct((R, N, D), f32),
                   jax.ShapeDtypeStruct((R, N, H), f32),
                   jax.ShapeDtypeStruct((R, D), f32)],
    )(s1p, c0p, lo2b, x, xF, w1a, w1b, w1c, mlp_b1, flo, fnt, featmask_b,
      node_emb, edge_emb, gcn0_W)

    em = _emk(A, B, src_main, dst_main, cvec, mlp_W2[:, 0])

    BE = 6400
    mm = pl.pallas_call(
        _t3_body,
        grid=(E // BE,),
        in_specs=[pl.BlockSpec((R, BE), lambda i: (0, i))],
        out_specs=pl.BlockSpec((8, 128), lambda i: (0, 0)),
        out_shape=jax.ShapeDtypeStruct((8, 128), f32),
    )(em)
    mn3 = mm[0, :R]
    inv3 = 1.0 / (-mm[1, :R] - mn3)
    mnmx16 = jnp.concatenate([mn3, inv3, jnp.zeros((16 - 2 * R,), f32)])

    wdegp = _degk(em, dst_main, mnmx16, zn).reshape(NC, R, N, 1)

    dinv, xw0p = pl.pallas_call(
        _t4_body,
        grid=(GRID,),
        in_specs=[_rows((2, R), (1,)), _rows((R,), (H,))],
        out_specs=[_rows((R,), (1,)), _rows((R,), (H,))],
        out_shape=[jax.ShapeDtypeStruct((R, N, 1), f32),
                   jax.ShapeDtypeStruct((R, N, H), f32)],
    )(wdegp, xw0)

    g0p = _gcnk(xw0p, src_main, dst_main, em, mnmx16, znh)

    xw1, xw1p = pl.pallas_call(
        _t5_body,
        grid=(GRID,),
        in_specs=[_rows((2, R), (H,)), _rows((R,), (H,)), _rows((R,), (1,)),
                  _full((R, H)), _full((R, H, H))],
        out_specs=[_rows((R,), (H,)), _rows((R,), (H,))],
        out_shape=[jax.ShapeDtypeStruct((R, N, H), f32),
                   jax.ShapeDtypeStruct((R, N, H), f32)],
    )(g0p, xw0, dinv, gcn0_b, gcn1_W)

    g1p = _gcnk(xw1p, src_main, dst_main, em, mnmx16, znh)

    out, x_temp = pl.pallas_call(
        _t6_body,
        grid=(GRID,),
        in_specs=[_rows((2, R), (H,)), _rows((R,), (H,)), _rows((R,), (1,)),
                  _full((R, H)), _full((R * H, OUT)), _full((OUT,))],
        out_specs=[_rows((), (OUT,)), _rows((), (R * H,))],
        out_shape=[jax.ShapeDtypeStruct((N, OUT), f32),
                   jax.ShapeDtypeStruct((N, R * H), f32)],
    )(g1p, xw1, dinv, gcn1_b, lin_W, lin_b)

    return out, x_temp


# speedup vs baseline: 13.2200x; 1.1755x over previous
import functools
import jax
import jax.numpy as jnp
from jax import lax
from jax.experimental import pallas as pl
from jax.experimental.pallas import tpu as pltpu, tpu_sc as plsc

N = 10000
E = 320000
R = 3
LT = 2
D = 128
H = 32
OUT = 2

NC, NS = 2, 16
NW = NC * NS
EW = E // NW
K = 1000
NCH = EW // K
KE = 400
NCHE = EW // KE
RPT = 624
TOFF = NS * RPT
TAIL = N - TOFF

BN = 400
GRID = N // BN

_MESH = plsc.VectorSubcoreMesh(core_axis_name="c", subcore_axis_name="s")
_CP_SC = pltpu.CompilerParams(use_tc_tiling_on_sc=False)


def _wid():
    return lax.axis_index("s") * NC + lax.axis_index("c")


def _make_seg3(S):
    @functools.partial(
        pl.kernel,
        out_type=(
            jax.ShapeDtypeStruct((NC, S, N, H), jnp.float32),
            jax.ShapeDtypeStruct((NC, S, N), jnp.float32),
        ),
        mesh=_MESH,
        compiler_params=_CP_SC,
        scratch_types=[
            pltpu.VMEM((K,), jnp.int32),
            pltpu.VMEM((K,), jnp.int32),
            pltpu.VMEM((K, H), jnp.float32),
            pltpu.VMEM((K,), jnp.float32),
            pltpu.VMEM_SHARED((S, N, H), jnp.float32),
            pltpu.VMEM_SHARED((S, N), jnp.float32),
            pltpu.SemaphoreType.DMA,
        ],
    )
    def seg3(tables, src, dst, znh, zn, out_vals, out_cnt, src_v, dst_v,
             rows_v, ones_v, acc_sh, cnt_sh, sem):
        sid = lax.axis_index("s")
        wid = _wid()
        one16 = jnp.ones((16,), jnp.float32)
        for i in range(K // 16):
            ones_v[pl.ds(16 * i, 16)] = one16
        for s in range(S):
            pltpu.sync_copy(znh.at[pl.ds(sid * RPT, RPT)],
                            acc_sh.at[s, pl.ds(sid * RPT, RPT)])
            pltpu.sync_copy(zn.at[pl.ds(sid * RPT, RPT)],
                            cnt_sh.at[s, pl.ds(sid * RPT, RPT)])

        @pl.when(sid == 0)
        def _():
            for s in range(S):
                pltpu.sync_copy(znh.at[pl.ds(TOFF, TAIL)],
                                acc_sh.at[s, pl.ds(TOFF, TAIL)])
                pltpu.sync_copy(zn.at[pl.ds(TOFF, TAIL)],
                                cnt_sh.at[s, pl.ds(TOFF, TAIL)])

        plsc.subcore_barrier()
        for s in range(S):
            def chunk(j, _):
                base = wid * EW + j * K
                pltpu.sync_copy(src.at[s, pl.ds(base, K)], src_v)
                pltpu.sync_copy(dst.at[s, pl.ds(base, K)], dst_v)
                pltpu.async_copy(tables.at[s].at[src_v], rows_v, sem).wait()
                pltpu.sync_copy(rows_v, acc_sh.at[s].at[dst_v], add=True)
                pltpu.sync_copy(ones_v, cnt_sh.at[s].at[dst_v], add=True)
                return 0
            lax.fori_loop(0, NCH, chunk, 0)
        plsc.subcore_barrier()
        cid = lax.axis_index("c")
        for s in range(S):
            pltpu.sync_copy(acc_sh.at[s, pl.ds(sid * RPT, RPT)],
                            out_vals.at[cid, s, pl.ds(sid * RPT, RPT)])
            pltpu.sync_copy(cnt_sh.at[s, pl.ds(sid * RPT, RPT)],
                            out_cnt.at[cid, s, pl.ds(sid * RPT, RPT)])

        @pl.when(sid == 0)
        def _():
            for s in range(S):
                pltpu.sync_copy(acc_sh.at[s, pl.ds(TOFF, TAIL)],
                                out_vals.at[cid, s, pl.ds(TOFF, TAIL)])
                pltpu.sync_copy(cnt_sh.at[s, pl.ds(TOFF, TAIL)],
                                out_cnt.at[cid, s, pl.ds(TOFF, TAIL)])

    return seg3


_seg3 = _make_seg3(R)


@functools.partial(
    pl.kernel,
    out_type=jax.ShapeDtypeStruct((R, E), jnp.float32),
    mesh=_MESH,
    compiler_params=_CP_SC,
    scratch_types=[
        pltpu.VMEM((KE,), jnp.int32),
        pltpu.VMEM((KE,), jnp.int32),
        pltpu.VMEM((KE, D), jnp.float32),
        pltpu.VMEM((KE, D), jnp.float32),
        pltpu.VMEM((R, D), jnp.float32),
        pltpu.VMEM((D,), jnp.float32),
        pltpu.VMEM((KE,), jnp.float32),
        pltpu.SemaphoreType.DMA,
    ],
)
def _emk(A, B, src, dst, cvec, w2, out, src_v, dst_v, ra_v, rb_v, c_v, w_v,
         em_v, sem):
    wid = _wid()
    pltpu.sync_copy(cvec, c_v)
    pltpu.sync_copy(w2, w_v)
    lane = lax.iota(jnp.int32, 16)
    ws = [w_v[pl.ds(16 * t, 16)] for t in range(D // 16)]
    for r in range(R):
        cs = [c_v[r, pl.ds(16 * t, 16)] for t in range(D // 16)]

        def chunk(j, _):
            base = wid * EW + j * KE
            pltpu.sync_copy(src.at[r, pl.ds(base, KE)], src_v)
            pltpu.sync_copy(dst.at[r, pl.ds(base, KE)], dst_v)
            pltpu.async_copy(A.at[r].at[src_v], ra_v, sem).wait()
            pltpu.async_copy(B.at[r].at[dst_v], rb_v, sem).wait()

            def group(g, _):
                vec = jnp.zeros((16,), jnp.float32)
                for i in range(16):
                    e = 16 * g + i
                    acc = jnp.zeros((16,), jnp.float32)
                    for t in range(D // 16):
                        a = ra_v[e, pl.ds(16 * t, 16)]
                        b = rb_v[e, pl.ds(16 * t, 16)]
                        acc = acc + jnp.maximum(a + b + cs[t], 0.0) * ws[t]
                    s01 = acc[0] + acc[1]
                    s23 = acc[2] + acc[3]
                    s45 = acc[4] + acc[5]
                    s67 = acc[6] + acc[7]
                    s89 = acc[8] + acc[9]
                    sab = acc[10] + acc[11]
                    scd = acc[12] + acc[13]
                    sef = acc[14] + acc[15]
                    tot = ((s01 + s23) + (s45 + s67)) + ((s89 + sab) + (scd + sef))
                    vec = jnp.where(lane == i, tot, vec)
                em_v[pl.ds(16 * g, 16)] = vec
                return 0

            lax.fori_loop(0, KE // 16, group, 0)
            pltpu.sync_copy(em_v, out.at[r, pl.ds(base, KE)])
            return 0

        lax.fori_loop(0, NCHE, chunk, 0)


@functools.partial(
    pl.kernel,
    out_type=jax.ShapeDtypeStruct((NC, R, N), jnp.float32),
    mesh=_MESH,
    compiler_params=_CP_SC,
    scratch_types=[
        pltpu.VMEM((K,), jnp.int32),
        pltpu.VMEM((K,), jnp.float32),
        pltpu.VMEM((K,), jnp.float32),
        pltpu.VMEM((16,), jnp.float32),
        pltpu.VMEM_SHARED((R, N), jnp.float32),
        pltpu.SemaphoreType.DMA,
    ],
)
def _degk(em, dst, mnmx, zn, out_deg, dst_v, em_v, ew_v, mm_v, cnt_sh, sem):
    sid = lax.axis_index("s")
    wid = _wid()
    pltpu.sync_copy(mnmx, mm_v)
    mm = mm_v[...]
    for r in range(R):
        pltpu.sync_copy(zn.at[pl.ds(sid * RPT, RPT)],
                        cnt_sh.at[r, pl.ds(sid * RPT, RPT)])

    @pl.when(sid == 0)
    def _():
        for r in range(R):
            pltpu.sync_copy(zn.at[pl.ds(TOFF, TAIL)],
                            cnt_sh.at[r, pl.ds(TOFF, TAIL)])

    plsc.subcore_barrier()
    for r in range(R):
        mn = mm[r]
        inv = mm[R + r]

        def chunk(j, _):
            base = wid * EW + j * K
            pltpu.sync_copy(dst.at[r, pl.ds(base, K)], dst_v)
            pltpu.sync_copy(em.at[r, pl.ds(base, K)], em_v)
            for g in range(K // 16):
                ew_v[pl.ds(16 * g, 16)] = (em_v[pl.ds(16 * g, 16)] - mn) * inv
            pltpu.sync_copy(ew_v, cnt_sh.at[r].at[dst_v], add=True)
            return 0

        lax.fori_loop(0, NCH, chunk, 0)
    plsc.subcore_barrier()
    cid = lax.axis_index("c")
    for r in range(R):
        pltpu.sync_copy(cnt_sh.at[r, pl.ds(sid * RPT, RPT)],
                        out_deg.at[cid, r, pl.ds(sid * RPT, RPT)])

    @pl.when(sid == 0)
    def _():
        for r in range(R):
            pltpu.sync_copy(cnt_sh.at[r, pl.ds(TOFF, TAIL)],
                            out_deg.at[cid, r, pl.ds(TOFF, TAIL)])


@functools.partial(
    pl.kernel,
    out_type=jax.ShapeDtypeStruct((NC, R, N, H), jnp.float32),
    mesh=_MESH,
    compiler_params=_CP_SC,
    scratch_types=[
        pltpu.VMEM((K,), jnp.int32),
        pltpu.VMEM((K,), jnp.int32),
        pltpu.VMEM((K,), jnp.float32),
        pltpu.VMEM((K, H), jnp.float32),
        pltpu.VMEM((16,), jnp.float32),
        pltpu.VMEM_SHARED((R, N, H), jnp.float32),
        pltpu.SemaphoreType.DMA,
    ],
)
def _gcnk(tables, src, dst, em, mnmx, znh, out_vals, src_v, dst_v, em_v,
          rows_v, mm_v, acc_sh, sem):
    sid = lax.axis_index("s")
    wid = _wid()
    pltpu.sync_copy(mnmx, mm_v)
    mm = mm_v[...]
    for r in range(R):
        pltpu.sync_copy(znh.at[pl.ds(sid * RPT, RPT)],
                        acc_sh.at[r, pl.ds(sid * RPT, RPT)])

    @pl.when(sid == 0)
    def _():
        for r in range(R):
            pltpu.sync_copy(znh.at[pl.ds(TOFF, TAIL)],
                            acc_sh.at[r, pl.ds(TOFF, TAIL)])

    plsc.subcore_barrier()
    for r in range(R):
        mn = mm[r]
        inv = mm[R + r]

        def chunk(j, _):
            base = wid * EW + j * K
            pltpu.sync_copy(src.at[r, pl.ds(base, K)], src_v)
            pltpu.sync_copy(dst.at[r, pl.ds(base, K)], dst_v)
            pltpu.sync_copy(em.at[r, pl.ds(base, K)], em_v)
            pltpu.async_copy(tables.at[r].at[src_v], rows_v, sem).wait()

            def group(g, _):
                ew16 = (em_v[pl.ds(16 * g, 16)] - mn) * inv
                for i in range(16):
                    e = 16 * g + i
                    w = ew16[i]
                    rows_v[e, pl.ds(0, 16)] = rows_v[e, pl.ds(0, 16)] * w
                    rows_v[e, pl.ds(16, 16)] = rows_v[e, pl.ds(16, 16)] * w
                return 0

            lax.fori_loop(0, K // 16, group, 0)
            pltpu.sync_copy(rows_v, acc_sh.at[r].at[dst_v], add=True)
            return 0

        lax.fori_loop(0, NCH, chunk, 0)
    plsc.subcore_barrier()
    cid = lax.axis_index("c")
    for r in range(R):
        pltpu.sync_copy(acc_sh.at[r, pl.ds(sid * RPT, RPT)],
                        out_vals.at[cid, r, pl.ds(sid * RPT, RPT)])

    @pl.when(sid == 0)
    def _():
        for r in range(R):
            pltpu.sync_copy(acc_sh.at[r, pl.ds(TOFF, TAIL)],
                            out_vals.at[cid, r, pl.ds(TOFF, TAIL)])


def _full(spec_shape):
    nd = len(spec_shape)
    return pl.BlockSpec(spec_shape, lambda i, _n=nd: (0,) * _n)


def _rows(shape_prefix, suffix):
    blk = tuple(shape_prefix) + (BN,) + tuple(suffix)
    npre = len(shape_prefix)
    nsuf = len(suffix)

    def imap(i, _p=npre, _s=nsuf):
        return (0,) * _p + (i,) + (0,) * _s

    return pl.BlockSpec(blk, imap)


def _t0_body(x_ref, w_ref, o_ref):
    o_ref[...] = jnp.dot(x_ref[...], w_ref[...],
                         preferred_element_type=jnp.float32)


def _t1_body(s0p, c0p, st0p, ct0p, st1p, ct1p, xw0r, xwtr0, xwtr1, b0l, tbl,
             gw, gb, w1l, w1r, b1l, l1l_o, lo2b_o):
    for r in range(R):
        cm = jnp.maximum(c0p[0, r] + c0p[1, r], 1.0)
        lo1 = jnp.maximum((s0p[0, r] + s0p[1, r]) / cm + b0l[r][None, :]
                          + xw0r[r], 0.0)
        ct0 = jnp.maximum(ct0p[0, r] + ct0p[1, r], 1.0)
        t0 = jnp.maximum((st0p[0, r] + st0p[1, r]) / ct0
                         + tbl[r, 0][None, :] + xwtr0[r], 0.0)
        ct1 = jnp.maximum(ct1p[0, r] + ct1p[1, r], 1.0)
        t1 = jnp.maximum((st1p[0, r] + st1p[1, r]) / ct1
                         + tbl[r, 1][None, :] + xwtr1[r], 0.0)
        g0 = jnp.dot(t0, gw[r, 0], preferred_element_type=jnp.float32) + gb[r, 0][None, :]
        g1 = jnp.dot(t1, gw[r, 1], preferred_element_type=jnp.float32) + gb[r, 1][None, :]
        m = jnp.maximum(g0, g1)
        e0 = jnp.exp(g0 - m)
        e1 = jnp.exp(g1 - m)
        tot = e0 + e1
        x_tree = t0 * (e0 / tot) + t1 * (e1 / tot)
        l1l_o[r] = jnp.dot(lo1, w1l[r], preferred_element_type=jnp.float32)
        lo2b_o[r] = (jnp.dot(lo1, w1r[r], preferred_element_type=jnp.float32)
                     + b1l[r][None, :] + x_tree)


def _t2_body(s1p, c0p, lo2b, x_ref, xf, w1a, w1b, w1c, b1, flo, fnt, fb, nemb,
             eemb, g0w, a_o, b_o, xw0_o, cvec_o):
    xblk = x_ref[...]
    for r in range(R):
        cm = jnp.maximum(c0p[0, r] + c0p[1, r], 1.0)
        lo = (s1p[0, r] + s1p[1, r]) / cm + lo2b[r]
        a_o[r] = jnp.dot(lo, w1a[...], preferred_element_type=jnp.float32)
        b_o[r] = jnp.dot(lo, w1b[...], preferred_element_type=jnp.float32)
        ntv = (jnp.dot(nemb[r][None, :], fnt[r],
                       preferred_element_type=jnp.float32) + fb[r][None, :])
        fm = (jnp.dot(lo, flo[r], preferred_element_type=jnp.float32)
              + xf[r] + ntv)
        ly = xblk * fm
        xw0_o[r] = jnp.dot(ly, g0w[r], preferred_element_type=jnp.float32)
        cvec_o[r] = (jnp.dot(eemb[r][None, :], w1c[...],
                             preferred_element_type=jnp.float32)[0]
                     + b1[...])


def _t3_body(em_ref, o_ref):
    i = pl.program_id(0)

    @pl.when(i == 0)
    def _():
        o_ref[...] = jnp.full((8, 128), jnp.inf, jnp.float32)

    lanes = lax.broadcasted_iota(jnp.int32, (1, 128), 1)
    for r in range(R):
        mnr = jnp.min(em_ref[r])
        mxn = jnp.min(-em_ref[r])
        o_ref[0:1, :] = jnp.minimum(o_ref[0:1, :],
                                    jnp.where(lanes == r, mnr, jnp.inf))
        o_ref[1:2, :] = jnp.minimum(o_ref[1:2, :],
                                    jnp.where(lanes == r, mxn, jnp.inf))


def _t4_body(wdegp, xw0, dinv_o, xw0p_o):
    deg = wdegp[0] + wdegp[1] + 1.0
    dinv = lax.rsqrt(deg)
    dinv_o[...] = dinv
    xw0p_o[...] = xw0[...] * dinv


def _t5_body(g0p, xw0, dinv, g0b, g1w, xw1_o, xw1p_o):
    for r in range(R):
        dv = dinv[r]
        h0 = jnp.maximum(dv * (g0p[0, r] + g0p[1, r])
                         + xw0[r] * (dv * dv) + g0b[r][None, :], 0.0)
        xw1 = jnp.dot(h0, g1w[r], preferred_element_type=jnp.float32)
        xw1_o[r] = xw1
        xw1p_o[r] = xw1 * dv


def _t6_body(g1p, xw1, dinv, g1b, lw, lb, out_o, xt_o):
    hs = []
    for r in range(R):
        dv = dinv[r]
        hs.append(dv * (g1p[0, r] + g1p[1, r]) + xw1[r] * (dv * dv)
                  + g1b[r][None, :])
    xt = jnp.concatenate(hs, axis=1)
    xt_o[...] = xt
    logits = jnp.dot(xt, lw[...], preferred_element_type=jnp.float32) + lb[...][None, :]
    logits = jnp.clip(logits, -1e10, 1e10)
    m = jnp.max(logits, axis=1, keepdims=True)
    s = logits - m
    lse = jnp.log(jnp.sum(jnp.exp(s), axis=1, keepdims=True))
    out_o[...] = s - lse


def kernel(x, edge_index, node_emb, edge_emb, sage0_Wl, sage0_bl, sage0_Wr,
           sage1_Wl, sage1_bl, sage1_Wr, tree_Wl, tree_bl, tree_Wr, gate_W,
           gate_b, featmask_W, featmask_b, mlp_W1, mlp_b1, mlp_W2, mlp_b2,
           gcn0_W, gcn0_b, gcn1_W, gcn1_b, lin_W, lin_b):
    f32 = jnp.float32
    x = x.astype(f32)
    ei = edge_index.astype(jnp.int32)
    src_main = ei[:, 0, 0]
    dst_main = ei[:, 0, 1]
    src_t0 = ei[:, 1, 0]
    dst_t0 = ei[:, 1, 1]
    src_t1 = ei[:, 2, 0]
    dst_t1 = ei[:, 2, 1]

    znh = jnp.zeros((N, H), f32)
    zn = jnp.zeros((N,), f32)

    w_s0l = sage0_Wl.transpose(1, 0, 2).reshape(D, R * H)
    w_trl = tree_Wl.transpose(2, 0, 1, 3).reshape(D, R * LT * H)
    w_s0r = sage0_Wr.transpose(1, 0, 2).reshape(D, R * H)
    w_trr = tree_Wr.transpose(2, 0, 1, 3).reshape(D, R * LT * H)
    w_fx = featmask_W[:, H:H + D, :].transpose(1, 0, 2).reshape(D, R * D)
    w_all = jnp.concatenate([w_s0l, w_trl, w_s0r, w_trr, w_fx], axis=1)
    CW = w_all.shape[1]

    t0 = pl.pallas_call(
        _t0_body,
        grid=(GRID,),
        in_specs=[_rows((), (D,)), _full((D, CW))],
        out_specs=_rows((), (CW,)),
        out_shape=jax.ShapeDtypeStruct((N, CW), f32),
    )(x, w_all)

    c = 0
    sage0_tab = t0[:, c:c + 96].reshape(N, R, H).transpose(1, 0, 2); c += 96
    tree_tab = t0[:, c:c + 192].reshape(N, R * LT, H).transpose(1, 0, 2); c += 192
    xW0r = t0[:, c:c + 96].reshape(N, R, H).transpose(1, 0, 2); c += 96
    xWtr = t0[:, c:c + 192].reshape(N, R * LT, H).transpose(1, 0, 2); c += 192
    xF = t0[:, c:c + 384].reshape(N, R, D).transpose(1, 0, 2)

    s0p, c0p = _seg3(sage0_tab, src_main, dst_main, znh, zn)
    st0p, ct0p = _seg3(tree_tab[0::2], src_t0, dst_t0, znh, zn)
    st1p, ct1p = _seg3(tree_tab[1::2], src_t1, dst_t1, znh, zn)
    c0p = c0p.reshape(NC, R, N, 1)
    ct0p = ct0p.reshape(NC, R, N, 1)
    ct1p = ct1p.reshape(NC, R, N, 1)

    l1l, lo2b = pl.pallas_call(
        _t1_body,
        grid=(GRID,),
        in_specs=[
            _rows((2, R), (H,)), _rows((2, R), (1,)),
            _rows((2, R), (H,)), _rows((2, R), (1,)),
            _rows((2, R), (H,)), _rows((2, R), (1,)),
            _rows((R,), (H,)), _rows((R,), (H,)), _rows((R,), (H,)),
            _full((R, H)), _full((R, LT, H)),
            _full((R, LT, H, 1)), _full((R, LT, 1)),
            _full((R, H, H)), _full((R, H, H)), _full((R, H)),
        ],
        out_specs=[_rows((R,), (H,)), _rows((R,), (H,))],
        out_shape=[jax.ShapeDtypeStruct((R, N, H), f32),
                   jax.ShapeDtypeStruct((R, N, H), f32)],
    )(s0p, c0p, st0p, ct0p, st1p, ct1p, xW0r, xWtr[0::2], xWtr[1::2],
      sage0_bl, tree_bl, gate_W, gate_b, sage1_Wl, sage1_Wr, sage1_bl)

    s1p, _unused = _seg3(l1l, src_main, dst_main, znh, zn)

    w1a = mlp_W1[0:H]
    w1b = mlp_W1[H:2 * H]
    w1c = mlp_W1[2 * H:3 * H]
    flo = featmask_W[:, 0:H, :]
    fnt = featmask_W[:, H + D:, :]

    A, B, xw0, cvec = pl.pallas_call(
        _t2_body,
        grid=(GRID,),
        in_specs=[
            _rows((2, R), (H,)), _rows((2, R), (1,)),
            _rows((R,), (H,)), _rows((), (D,)), _rows((R,), (D,)),
            _full((H, D)), _full((H, D)), _full((H, D)), _full((D,)),
            _full((R, H, D)), _full((R, H, D)), _full((R, D)),
            _full((R, H)), _full((R, H)), _full((R, D, H)),
        ],
        out_specs=[_rows((R,), (D,)), _rows((R,), (D,)), _rows((R,), (H,)),
                   _full((R, D))],
        out_shape=[jax.ShapeDtypeStruct((R, N, D), f32),
                   jax.ShapeDtypeStruct((R, N, D), f32),
                   jax.ShapeDtypeStruct((R, N, H), f32),
                   jax.ShapeDtypeStruct((R, D), f32)],
    )(s1p, c0p, lo2b, x, xF, w1a, w1b, w1c, mlp_b1, flo, fnt, featmask_b,
      node_emb, edge_emb, gcn0_W)

    em = _emk(A, B, src_main, dst_main, cvec, mlp_W2[:, 0])

    BE = 6400
    mm = pl.pallas_call(
        _t3_body,
        grid=(E // BE,),
        in_specs=[pl.BlockSpec((R, BE), lambda i: (0, i))],
        out_specs=pl.BlockSpec((8, 128), lambda i: (0, 0)),
        out_shape=jax.ShapeDtypeStruct((8, 128), f32),
    )(em)
    mn3 = mm[0, :R]
    inv3 = 1.0 / (-mm[1, :R] - mn3)
    mnmx16 = jnp.concatenate([mn3, inv3, jnp.zeros((16 - 2 * R,), f32)])

    wdegp = _degk(em, dst_main, mnmx16, zn).reshape(NC, R, N, 1)

    dinv, xw0p = pl.pallas_call(
        _t4_body,
        grid=(GRID,),
        in_specs=[_rows((2, R), (1,)), _rows((R,), (H,))],
        out_specs=[_rows((R,), (1,)), _rows((R,), (H,))],
        out_shape=[jax.ShapeDtypeStruct((R, N, 1), f32),
                   jax.ShapeDtypeStruct((R, N, H), f32)],
    )(wdegp, xw0)

    g0p = _gcnk(xw0p, src_main, dst_main, em, mnmx16, znh)

    xw1, xw1p = pl.pallas_call(
        _t5_body,
        grid=(GRID,),
        in_specs=[_rows((2, R), (H,)), _rows((R,), (H,)), _rows((R,), (1,)),
                  _full((R, H)), _full((R, H, H))],
        out_specs=[_rows((R,), (H,)), _rows((R,), (H,))],
        out_shape=[jax.ShapeDtypeStruct((R, N, H), f32),
                   jax.ShapeDtypeStruct((R, N, H), f32)],
    )(g0p, xw0, dinv, gcn0_b, gcn1_W)

    g1p = _gcnk(xw1p, src_main, dst_main, em, mnmx16, znh)

    out, x_temp = pl.pallas_call(
        _t6_body,
        grid=(GRID,),
        in_specs=[_rows((2, R), (H,)), _rows((R,), (H,)), _rows((R,), (1,)),
                  _full((R, H)), _full((R * H, OUT)), _full((OUT,))],
        out_specs=[_rows((), (OUT,)), _rows((), (R * H,))],
        out_shape=[jax.ShapeDtypeStruct((N, OUT), f32),
                   jax.ShapeDtypeStruct((N, R * H), f32)],
    )(g1p, xw1, dinv, gcn1_b, lin_W, lin_b)

    return out, x_temp
